# baseline (speedup 1.0000x reference)
import dataclasses

import jax
import jax.numpy as jnp
from jax import lax
from jax.experimental import pallas as pl
from jax.experimental.pallas import tpu as pltpu
from jax.experimental.pallas import tpu_sc as plsc

_SAMPLES_PER_STEP = 4
_LANES = 16


def _sc_gather_body(ts_hbm, sab_hbm, somab_hbm, a_hbm, b_hbm,
                    ts_v, sa_v, so_v, a_v, b_v):
    cid = lax.axis_index("c")
    sid = lax.axis_index("s")

    @pl.when(jnp.logical_and(cid == 0, sid == 0))
    def _():
        pltpu.sync_copy(ts_hbm, ts_v)
        pltpu.sync_copy(sab_hbm, sa_v)
        pltpu.sync_copy(somab_hbm, so_v)
        for k in range(ts_v.shape[0] // _LANES):
            sl = pl.ds(k * _LANES, _LANES)
            idx = ts_v[sl]
            a_v[sl] = plsc.load_gather(sa_v, [idx])
            b_v[sl] = plsc.load_gather(so_v, [idx])
        pltpu.sync_copy(a_v, a_hbm)
        pltpu.sync_copy(b_v, b_hbm)


def _sc_gather(ts, sab, somab):
    B = ts.shape[0]
    n_steps = sab.shape[0]
    mesh = plsc.VectorSubcoreMesh(core_axis_name="c", subcore_axis_name="s")
    f32 = jnp.float32
    cp = pltpu.CompilerParams()
    if "needs_layout_passes" in pltpu.CompilerParams.__dataclass_fields__:
        cp = dataclasses.replace(cp, needs_layout_passes=False)
    gather = pl.kernel(
        _sc_gather_body,
        out_type=(jax.ShapeDtypeStruct((B,), f32),
                  jax.ShapeDtypeStruct((B,), f32)),
        mesh=mesh,
        scratch_types=[
            pltpu.VMEM((B,), jnp.int32),
            pltpu.VMEM((n_steps,), f32),
            pltpu.VMEM((n_steps,), f32),
            pltpu.VMEM((B,), f32),
            pltpu.VMEM((B,), f32),
        ],
        compiler_params=cp,
    )
    return gather(ts, sab, somab)


def _tc_body(a_ref, b_ref, x_ref, n_ref, o_ref):
    i = pl.program_id(0)
    for j in range(_SAMPLES_PER_STEP):
        a = a_ref[i * _SAMPLES_PER_STEP + j]
        b = b_ref[i * _SAMPLES_PER_STEP + j]
        o_ref[j] = a * x_ref[j] + b * n_ref[j]


def kernel(x0, noise, time_steps, sqrt_alpha_bars, sqrt_one_minus_alpha_bars):
    B, C, H, W = x0.shape
    ts = time_steps.astype(jnp.int32)
    a_t, b_t = _sc_gather(ts, sqrt_alpha_bars, sqrt_one_minus_alpha_bars)
    out = pl.pallas_call(
        _tc_body,
        grid=(B // _SAMPLES_PER_STEP,),
        in_specs=[
            pl.BlockSpec(memory_space=pltpu.SMEM),
            pl.BlockSpec(memory_space=pltpu.SMEM),
            pl.BlockSpec((_SAMPLES_PER_STEP, C, H, W), lambda i: (i, 0, 0, 0)),
            pl.BlockSpec((_SAMPLES_PER_STEP, C, H, W), lambda i: (i, 0, 0, 0)),
        ],
        out_specs=pl.BlockSpec((_SAMPLES_PER_STEP, C, H, W), lambda i: (i, 0, 0, 0)),
        out_shape=jax.ShapeDtypeStruct((B, C, H, W), x0.dtype),
    )(a_t, b_t, x0, noise)
    return out


# speedup vs baseline: 1.0128x; 1.0128x over previous
import dataclasses

import jax
import jax.numpy as jnp
from jax import lax
from jax.experimental import pallas as pl
from jax.experimental.pallas import tpu as pltpu
from jax.experimental.pallas import tpu_sc as plsc

_SAMPLES_PER_STEP = 4
_LANES = 16


def _sc_gather_body(ts_hbm, tabs_hbm, ab_hbm,
                    ts_v, tabs_v, ab_v, sem0, sem1):
    cid = lax.axis_index("c")
    sid = lax.axis_index("s")
    B = ts_v.shape[0]

    @pl.when(jnp.logical_and(cid == 0, sid == 0))
    def _():
        cp0 = pltpu.async_copy(ts_hbm, ts_v, sem0)
        cp1 = pltpu.async_copy(tabs_hbm, tabs_v, sem1)
        cp0.wait()
        cp1.wait()
        n_steps = tabs_v.shape[0] // 2
        for k in range(B // _LANES):
            sl = pl.ds(k * _LANES, _LANES)
            idx = ts_v[sl]
            ab_v[pl.ds(k * _LANES, _LANES)] = plsc.load_gather(tabs_v, [idx])
            ab_v[pl.ds(B + k * _LANES, _LANES)] = plsc.load_gather(
                tabs_v, [idx + n_steps])
        pltpu.sync_copy(ab_v, ab_hbm)


def _sc_gather(ts, sab, somab):
    B = ts.shape[0]
    n_steps = sab.shape[0]
    tabs = jnp.concatenate([sab, somab])
    mesh = plsc.VectorSubcoreMesh(core_axis_name="c", subcore_axis_name="s")
    f32 = jnp.float32
    cp = pltpu.CompilerParams()
    if "needs_layout_passes" in pltpu.CompilerParams.__dataclass_fields__:
        cp = dataclasses.replace(cp, needs_layout_passes=False)
    gather = pl.kernel(
        _sc_gather_body,
        out_type=jax.ShapeDtypeStruct((2 * B,), f32),
        mesh=mesh,
        scratch_types=[
            pltpu.VMEM((B,), jnp.int32),
            pltpu.VMEM((2 * n_steps,), f32),
            pltpu.VMEM((2 * B,), f32),
            pltpu.SemaphoreType.DMA,
            pltpu.SemaphoreType.DMA,
        ],
        compiler_params=cp,
    )
    return gather(ts, tabs)


def _tc_body(ab_ref, x_ref, n_ref, o_ref):
    i = pl.program_id(0)
    B = ab_ref.shape[0] // 2
    for j in range(_SAMPLES_PER_STEP):
        s = i * _SAMPLES_PER_STEP + j
        a = ab_ref[s]
        b = ab_ref[B + s]
        o_ref[j] = a * x_ref[j] + b * n_ref[j]


def kernel(x0, noise, time_steps, sqrt_alpha_bars, sqrt_one_minus_alpha_bars):
    B, C, H, W = x0.shape
    ts = time_steps.astype(jnp.int32)
    ab = _sc_gather(ts, sqrt_alpha_bars, sqrt_one_minus_alpha_bars)
    out = pl.pallas_call(
        _tc_body,
        grid=(B // _SAMPLES_PER_STEP,),
        in_specs=[
            pl.BlockSpec(memory_space=pltpu.SMEM),
            pl.BlockSpec((_SAMPLES_PER_STEP, C, H, W), lambda i: (i, 0, 0, 0)),
            pl.BlockSpec((_SAMPLES_PER_STEP, C, H, W), lambda i: (i, 0, 0, 0)),
        ],
        out_specs=pl.BlockSpec((_SAMPLES_PER_STEP, C, H, W), lambda i: (i, 0, 0, 0)),
        out_shape=jax.ShapeDtypeStruct((B, C, H, W), x0.dtype),
    )(ab, x0, noise)
    return out


# speedup vs baseline: 1.4200x; 1.4020x over previous
import jax
import jax.numpy as jnp
from jax.experimental import pallas as pl
from jax.experimental.pallas import tpu as pltpu


_NBUF = 4


def _ddpm_body(ts_ref, sab_ref, somab_ref, x_hbm, n_hbm, o_hbm,
               xb, nb, ob, in_sems, out_sems):
    B = x_hbm.shape[0]

    def start_in(i):
        s = i % _NBUF
        pltpu.make_async_copy(x_hbm.at[i], xb.at[s], in_sems.at[s, 0]).start()
        pltpu.make_async_copy(n_hbm.at[i], nb.at[s], in_sems.at[s, 1]).start()

    def wait_in(i):
        s = i % _NBUF
        pltpu.make_async_copy(x_hbm.at[i], xb.at[s], in_sems.at[s, 0]).wait()
        pltpu.make_async_copy(n_hbm.at[i], nb.at[s], in_sems.at[s, 1]).wait()

    for i in range(_NBUF - 1):
        start_in(i)
    for i in range(B):
        s = i % _NBUF
        if i + _NBUF - 1 < B:
            start_in(i + _NBUF - 1)
        wait_in(i)
        if i >= _NBUF:
            pltpu.make_async_copy(
                ob.at[s], o_hbm.at[i - _NBUF], out_sems.at[s]).wait()
        t = ts_ref[i]
        a = sab_ref[t]
        b = somab_ref[t]
        ob[s] = a * xb[s] + b * nb[s]
        pltpu.make_async_copy(ob.at[s], o_hbm.at[i], out_sems.at[s]).start()
    for i in range(B - _NBUF, B):
        s = i % _NBUF
        pltpu.make_async_copy(ob.at[s], o_hbm.at[i], out_sems.at[s]).wait()


def kernel(x0, noise, time_steps, sqrt_alpha_bars, sqrt_one_minus_alpha_bars):
    B, C, H, W = x0.shape
    ts = time_steps.astype(jnp.int32)
    out = pl.pallas_call(
        _ddpm_body,
        in_specs=[
            pl.BlockSpec(memory_space=pltpu.SMEM),
            pl.BlockSpec(memory_space=pltpu.SMEM),
            pl.BlockSpec(memory_space=pltpu.SMEM),
            pl.BlockSpec(memory_space=pl.ANY),
            pl.BlockSpec(memory_space=pl.ANY),
        ],
        out_specs=pl.BlockSpec(memory_space=pl.ANY),
        out_shape=jax.ShapeDtypeStruct((B, C, H, W), x0.dtype),
        scratch_shapes=[
            pltpu.VMEM((_NBUF, C, H, W), jnp.float32),
            pltpu.VMEM((_NBUF, C, H, W), jnp.float32),
            pltpu.VMEM((_NBUF, C, H, W), jnp.float32),
            pltpu.SemaphoreType.DMA((_NBUF, 2)),
            pltpu.SemaphoreType.DMA((_NBUF,)),
        ],
    )(ts, sqrt_alpha_bars, sqrt_one_minus_alpha_bars, x0, noise)
    return out


# speedup vs baseline: 1.4224x; 1.0017x over previous
import jax
import jax.numpy as jnp
from jax.experimental import pallas as pl
from jax.experimental.pallas import tpu as pltpu


_NBUF = 8


def _ddpm_body(ts_ref, sab_ref, somab_ref, x_hbm, n_hbm, o_hbm,
               xb, nb, ob, in_sems, out_sems):
    B = x_hbm.shape[0]

    def start_in(i):
        s = i % _NBUF
        pltpu.make_async_copy(x_hbm.at[i], xb.at[s], in_sems.at[s, 0]).start()
        pltpu.make_async_copy(n_hbm.at[i], nb.at[s], in_sems.at[s, 1]).start()

    def wait_in(i):
        s = i % _NBUF
        pltpu.make_async_copy(x_hbm.at[i], xb.at[s], in_sems.at[s, 0]).wait()
        pltpu.make_async_copy(n_hbm.at[i], nb.at[s], in_sems.at[s, 1]).wait()

    for i in range(_NBUF - 1):
        start_in(i)
    for i in range(B):
        s = i % _NBUF
        if i + _NBUF - 1 < B:
            start_in(i + _NBUF - 1)
        wait_in(i)
        if i >= _NBUF:
            pltpu.make_async_copy(
                ob.at[s], o_hbm.at[i - _NBUF], out_sems.at[s]).wait()
        t = ts_ref[i]
        a = sab_ref[t]
        b = somab_ref[t]
        ob[s] = a * xb[s] + b * nb[s]
        pltpu.make_async_copy(ob.at[s], o_hbm.at[i], out_sems.at[s]).start()
    for i in range(B - _NBUF, B):
        s = i % _NBUF
        pltpu.make_async_copy(ob.at[s], o_hbm.at[i], out_sems.at[s]).wait()


def kernel(x0, noise, time_steps, sqrt_alpha_bars, sqrt_one_minus_alpha_bars):
    B, C, H, W = x0.shape
    ts = time_steps.astype(jnp.int32)
    out = pl.pallas_call(
        _ddpm_body,
        in_specs=[
            pl.BlockSpec(memory_space=pltpu.SMEM),
            pl.BlockSpec(memory_space=pltpu.SMEM),
            pl.BlockSpec(memory_space=pltpu.SMEM),
            pl.BlockSpec(memory_space=pl.ANY),
            pl.BlockSpec(memory_space=pl.ANY),
        ],
        out_specs=pl.BlockSpec(memory_space=pl.ANY),
        out_shape=jax.ShapeDtypeStruct((B, C, H, W), x0.dtype),
        scratch_shapes=[
            pltpu.VMEM((_NBUF, C, H, W), jnp.float32),
            pltpu.VMEM((_NBUF, C, H, W), jnp.float32),
            pltpu.VMEM((_NBUF, C, H, W), jnp.float32),
            pltpu.SemaphoreType.DMA((_NBUF, 2)),
            pltpu.SemaphoreType.DMA((_NBUF,)),
        ],
    )(ts, sqrt_alpha_bars, sqrt_one_minus_alpha_bars, x0, noise)
    return out


# speedup vs baseline: 1.4275x; 1.0036x over previous
import jax
import jax.numpy as jnp
from jax.experimental import pallas as pl
from jax.experimental.pallas import tpu as pltpu


_NBUF = 6
_ITEM = 2


def _ddpm_body(ts_ref, sab_ref, somab_ref, x_hbm, n_hbm, o_hbm,
               xb, nb, ob, in_sems, out_sems):
    n_items = x_hbm.shape[0] // _ITEM

    def start_in(i):
        s = i % _NBUF
        sl = pl.ds(i * _ITEM, _ITEM)
        pltpu.make_async_copy(x_hbm.at[sl], xb.at[s], in_sems.at[s, 0]).start()
        pltpu.make_async_copy(n_hbm.at[sl], nb.at[s], in_sems.at[s, 1]).start()

    def wait_in(i):
        s = i % _NBUF
        sl = pl.ds(i * _ITEM, _ITEM)
        pltpu.make_async_copy(x_hbm.at[sl], xb.at[s], in_sems.at[s, 0]).wait()
        pltpu.make_async_copy(n_hbm.at[sl], nb.at[s], in_sems.at[s, 1]).wait()

    def out_copy(i):
        s = i % _NBUF
        sl = pl.ds(i * _ITEM, _ITEM)
        return pltpu.make_async_copy(ob.at[s], o_hbm.at[sl], out_sems.at[s])

    for i in range(_NBUF - 1):
        start_in(i)
    for i in range(n_items):
        s = i % _NBUF
        if i + _NBUF - 1 < n_items:
            start_in(i + _NBUF - 1)
        wait_in(i)
        if i >= _NBUF:
            out_copy(i - _NBUF).wait()
        for j in range(_ITEM):
            t = ts_ref[i * _ITEM + j]
            a = sab_ref[t]
            b = somab_ref[t]
            ob[s, j] = a * xb[s, j] + b * nb[s, j]
        out_copy(i).start()
    for i in range(n_items - _NBUF, n_items):
        out_copy(i).wait()


def kernel(x0, noise, time_steps, sqrt_alpha_bars, sqrt_one_minus_alpha_bars):
    B, C, H, W = x0.shape
    ts = time_steps.astype(jnp.int32)
    out = pl.pallas_call(
        _ddpm_body,
        in_specs=[
            pl.BlockSpec(memory_space=pltpu.SMEM),
            pl.BlockSpec(memory_space=pltpu.SMEM),
            pl.BlockSpec(memory_space=pltpu.SMEM),
            pl.BlockSpec(memory_space=pl.ANY),
            pl.BlockSpec(memory_space=pl.ANY),
        ],
        out_specs=pl.BlockSpec(memory_space=pl.ANY),
        out_shape=jax.ShapeDtypeStruct((B, C, H, W), x0.dtype),
        scratch_shapes=[
            pltpu.VMEM((_NBUF, _ITEM, C, H, W), jnp.float32),
            pltpu.VMEM((_NBUF, _ITEM, C, H, W), jnp.float32),
            pltpu.VMEM((_NBUF, _ITEM, C, H, W), jnp.float32),
            pltpu.SemaphoreType.DMA((_NBUF, 2)),
            pltpu.SemaphoreType.DMA((_NBUF,)),
        ],
    )(ts, sqrt_alpha_bars, sqrt_one_minus_alpha_bars, x0, noise)
    return out


# speedup vs baseline: 1.4435x; 1.0112x over previous
import jax
import jax.numpy as jnp
from jax.experimental import pallas as pl
from jax.experimental.pallas import tpu as pltpu


_SAMPLES_PER_STEP = 4


def _ddpm_body(ts_ref, sab_ref, somab_ref, x_ref, n_ref, o_ref):
    i = pl.program_id(0)
    for j in range(_SAMPLES_PER_STEP):
        t = ts_ref[i * _SAMPLES_PER_STEP + j]
        a = sab_ref[t]
        b = somab_ref[t]
        o_ref[j] = a * x_ref[j] + b * n_ref[j]


def kernel(x0, noise, time_steps, sqrt_alpha_bars, sqrt_one_minus_alpha_bars):
    B, C, H, W = x0.shape
    ts = time_steps.astype(jnp.int32)
    out = pl.pallas_call(
        _ddpm_body,
        grid=(B // _SAMPLES_PER_STEP,),
        in_specs=[
            pl.BlockSpec(memory_space=pltpu.SMEM),
            pl.BlockSpec(memory_space=pltpu.SMEM),
            pl.BlockSpec(memory_space=pltpu.SMEM),
            pl.BlockSpec((_SAMPLES_PER_STEP, C, H, W), lambda i: (i, 0, 0, 0)),
            pl.BlockSpec((_SAMPLES_PER_STEP, C, H, W), lambda i: (i, 0, 0, 0)),
        ],
        out_specs=pl.BlockSpec((_SAMPLES_PER_STEP, C, H, W), lambda i: (i, 0, 0, 0)),
        out_shape=jax.ShapeDtypeStruct((B, C, H, W), x0.dtype),
    )(ts, sqrt_alpha_bars, sqrt_one_minus_alpha_bars, x0, noise)
    return out
